# baseline (speedup 1.0000x reference)
import functools

import jax
import jax.numpy as jnp
from jax import lax
from jax.experimental import pallas as pl
from jax.experimental.pallas import tpu as pltpu
from jax.experimental.pallas import tpu_sc as plsc

N_CLS = 1000
N_CTX = 16
CTX_DIM = 512
BATCH = 4096
SUF_LEN = 60
TOT_LEN = 1 + N_CTX + SUF_LEN

_D = CTX_DIM
_CTX_W = N_CTX * _D

_NC = 2
_NS = 16
_NW = _NC * _NS
_BPW = BATCH // _NW
_CH = 8
_NCHUNK = _BPW // _CH


def _sc_gather_kernel():
    mesh = plsc.VectorSubcoreMesh(core_axis_name="c", subcore_axis_name="s")

    @functools.partial(
        pl.kernel,
        mesh=mesh,
        out_type=jax.ShapeDtypeStruct((BATCH, _CTX_W), jnp.float32),
        scratch_types=[
            pltpu.VMEM((_BPW,), jnp.int32),
            pltpu.VMEM((_CH, _CTX_W), jnp.float32),
            pltpu.SemaphoreType.DMA,
        ],
    )
    def k(label_hbm, ctx_hbm, out_hbm, idx_v, buf, sem):
        wid = lax.axis_index("s") * _NC + lax.axis_index("c")
        base = wid * _BPW

        pltpu.sync_copy(label_hbm.at[pl.ds(base, _BPW)], idx_v.at[...])

        def body(c, carry):
            pltpu.async_copy(
                ctx_hbm.at[idx_v.at[pl.ds(c * _CH, _CH)]], buf, sem).wait()
            pltpu.sync_copy(buf.at[...], out_hbm.at[pl.ds(base + c * _CH, _CH)])
            return carry

        lax.fori_loop(0, _NCHUNK, body, 0)

    return k


_BB = 32


def _tc_assemble(pref_ref, gath_ref, suf_ref, out_ref):
    out_ref[:, 0:1, :] = pref_ref[...]
    out_ref[:, 1:1 + N_CTX, :] = gath_ref[...]
    out_ref[:, 1 + N_CTX:, :] = suf_ref[...]


def _tc_assemble_call(prefix, gathered3, suffix):
    return pl.pallas_call(
        _tc_assemble,
        grid=(BATCH // _BB,),
        in_specs=[
            pl.BlockSpec((_BB, 1, _D), lambda i: (i, 0, 0)),
            pl.BlockSpec((_BB, N_CTX, _D), lambda i: (i, 0, 0)),
            pl.BlockSpec((_BB, SUF_LEN, _D), lambda i: (i, 0, 0)),
        ],
        out_specs=pl.BlockSpec((_BB, TOT_LEN, _D), lambda i: (i, 0, 0)),
        out_shape=jax.ShapeDtypeStruct((BATCH, TOT_LEN, _D), jnp.float32),
    )(prefix, gathered3, suffix)


def kernel(label, prefix, suffix, ctx):
    label32 = label.astype(jnp.int32).reshape(BATCH)
    ctx2 = ctx.reshape(N_CLS, _CTX_W)
    gathered = _sc_gather_kernel()(label32, ctx2)
    gathered3 = gathered.reshape(BATCH, N_CTX, _D)
    return _tc_assemble_call(prefix, gathered3, suffix)


# speedup vs baseline: 1.0027x; 1.0027x over previous
import functools

import jax
import jax.numpy as jnp
from jax import lax
from jax.experimental import pallas as pl
from jax.experimental.pallas import tpu as pltpu
from jax.experimental.pallas import tpu_sc as plsc

N_CLS = 1000
N_CTX = 16
CTX_DIM = 512
BATCH = 4096
SUF_LEN = 60
TOT_LEN = 1 + N_CTX + SUF_LEN

_D = CTX_DIM
_CTX_W = N_CTX * _D

_NC = 2
_NS = 16
_NW = _NC * _NS
_BPW = BATCH // _NW
_CH = 8
_NCHUNK = _BPW // _CH


def _sc_gather_kernel():
    mesh = plsc.VectorSubcoreMesh(core_axis_name="c", subcore_axis_name="s")

    @functools.partial(
        pl.kernel,
        mesh=mesh,
        out_type=jax.ShapeDtypeStruct((BATCH, _CTX_W), jnp.float32),
        scratch_types=[
            pltpu.VMEM((_BPW,), jnp.int32),
            pltpu.VMEM((_CH, _CTX_W), jnp.float32),
            pltpu.SemaphoreType.DMA,
        ],
    )
    def k(label_hbm, ctx_hbm, out_hbm, idx_v, buf, sem):
        wid = lax.axis_index("s") * _NC + lax.axis_index("c")
        base = wid * _BPW

        pltpu.sync_copy(label_hbm.at[pl.ds(base, _BPW)], idx_v.at[...])

        def body(c, carry):
            pltpu.async_copy(
                ctx_hbm.at[idx_v.at[pl.ds(c * _CH, _CH)]], buf, sem).wait()
            pltpu.sync_copy(buf.at[...], out_hbm.at[pl.ds(base + c * _CH, _CH)])
            return carry

        lax.fori_loop(0, _NCHUNK, body, 0)

    return k


_BB = 16
_NBUF = 8
_NBLK = BATCH // _BB
_NGRP = _NBLK // _NBUF


def _tc_assemble(pref_hbm, gath_hbm, suf_hbm, out_hbm, *scratch):
    pbuf = scratch[0:_NBUF]
    gbuf = scratch[_NBUF:2 * _NBUF]
    sbuf = scratch[2 * _NBUF:3 * _NBUF]
    obuf = scratch[3 * _NBUF:4 * _NBUF]
    semi = scratch[4 * _NBUF]
    semo = scratch[4 * _NBUF + 1]

    def in_copies(b, k):
        r = b * _BB
        return (
            pltpu.make_async_copy(pref_hbm.at[pl.ds(r, _BB)], pbuf[k], semi.at[k, 0]),
            pltpu.make_async_copy(gath_hbm.at[pl.ds(r, _BB)], gbuf[k], semi.at[k, 1]),
            pltpu.make_async_copy(suf_hbm.at[pl.ds(r, _BB)], sbuf[k], semi.at[k, 2]),
        )

    def out_copy(b, k):
        return pltpu.make_async_copy(obuf[k], out_hbm.at[pl.ds(b * _BB, _BB)], semo.at[k])

    for k in range(_NBUF):
        for cp in in_copies(k, k):
            cp.start()

    def turn(g, carry):
        for k in range(_NBUF):
            b = g * _NBUF + k
            for cp in in_copies(b, k):
                cp.wait()

            @pl.when(g > 0)
            def _():
                out_copy(b, k).wait()

            obuf[k][:, 0:1, :] = pbuf[k][...]
            obuf[k][:, 1:1 + N_CTX, :] = gbuf[k][...]
            obuf[k][:, 1 + N_CTX:, :] = sbuf[k][...]
            out_copy(b, k).start()

            @pl.when(b + _NBUF < _NBLK)
            def _():
                for cp in in_copies(b + _NBUF, k):
                    cp.start()

        return carry

    lax.fori_loop(0, _NGRP, turn, 0)

    for k in range(_NBUF):
        out_copy(_NBLK - _NBUF + k, k).wait()


def _tc_assemble_call(prefix, gathered3, suffix):
    scratch = (
        [pltpu.VMEM((_BB, 1, _D), jnp.float32)] * _NBUF
        + [pltpu.VMEM((_BB, N_CTX, _D), jnp.float32)] * _NBUF
        + [pltpu.VMEM((_BB, SUF_LEN, _D), jnp.float32)] * _NBUF
        + [pltpu.VMEM((_BB, TOT_LEN, _D), jnp.float32)] * _NBUF
        + [pltpu.SemaphoreType.DMA((_NBUF, 3)), pltpu.SemaphoreType.DMA((_NBUF,))]
    )
    return pl.pallas_call(
        _tc_assemble,
        in_specs=[
            pl.BlockSpec(memory_space=pl.ANY),
            pl.BlockSpec(memory_space=pl.ANY),
            pl.BlockSpec(memory_space=pl.ANY),
        ],
        out_specs=pl.BlockSpec(memory_space=pl.ANY),
        out_shape=jax.ShapeDtypeStruct((BATCH, TOT_LEN, _D), jnp.float32),
        scratch_shapes=scratch,
    )(prefix, gathered3, suffix)


def kernel(label, prefix, suffix, ctx):
    label32 = label.astype(jnp.int32).reshape(BATCH)
    ctx2 = ctx.reshape(N_CLS, _CTX_W)
    gathered = _sc_gather_kernel()(label32, ctx2)
    gathered3 = gathered.reshape(BATCH, N_CTX, _D)
    return _tc_assemble_call(prefix, gathered3, suffix)


# speedup vs baseline: 2.6180x; 2.6109x over previous
import functools

import jax
import jax.numpy as jnp
from jax import lax
from jax.experimental import pallas as pl
from jax.experimental.pallas import tpu as pltpu
from jax.experimental.pallas import tpu_sc as plsc

N_CLS = 1000
N_CTX = 16
CTX_DIM = 512
BATCH = 4096
SUF_LEN = 60
TOT_LEN = 1 + N_CTX + SUF_LEN

_D = CTX_DIM
_CTX_W = N_CTX * _D

_NC = 2
_NS = 16
_NW = _NC * _NS
_BPW = BATCH // _NW
_CH = 8
_NCHUNK = _BPW // _CH


def _sc_gather_kernel():
    mesh = plsc.VectorSubcoreMesh(core_axis_name="c", subcore_axis_name="s")

    @functools.partial(
        pl.kernel,
        mesh=mesh,
        out_type=jax.ShapeDtypeStruct((BATCH, _CTX_W), jnp.float32),
        scratch_types=[
            pltpu.VMEM((_BPW,), jnp.int32),
            pltpu.VMEM((_CH, _CTX_W), jnp.float32),
            pltpu.SemaphoreType.DMA,
        ],
    )
    def k(label_hbm, ctx_hbm, out_hbm, idx_v, buf, sem):
        wid = lax.axis_index("s") * _NC + lax.axis_index("c")
        base = wid * _BPW

        pltpu.sync_copy(label_hbm.at[pl.ds(base, _BPW)], idx_v.at[...])

        def body(c, carry):
            pltpu.async_copy(
                ctx_hbm.at[idx_v.at[pl.ds(c * _CH, _CH)]], buf, sem).wait()
            pltpu.sync_copy(buf.at[...], out_hbm.at[pl.ds(base + c * _CH, _CH)])
            return carry

        lax.fori_loop(0, _NCHUNK, body, 0)

    return k


_BB = 32


def _tc_assemble(pref_ref, gath_ref, suf_ref, out_ref):
    out_ref[0, :, :] = pref_ref[:, 0, :]
    for t in range(N_CTX):
        out_ref[1 + t, :, :] = gath_ref[:, t * _D:(t + 1) * _D]
    out_ref[1 + N_CTX:, :, :] = suf_ref[...]


def _tc_assemble_call(prefix, gathered, suffix_t):
    return pl.pallas_call(
        _tc_assemble,
        grid=(BATCH // _BB,),
        in_specs=[
            pl.BlockSpec((_BB, 1, _D), lambda i: (i, 0, 0)),
            pl.BlockSpec((_BB, _CTX_W), lambda i: (i, 0)),
            pl.BlockSpec((SUF_LEN, _BB, _D), lambda i: (0, i, 0)),
        ],
        out_specs=pl.BlockSpec((TOT_LEN, _BB, _D), lambda i: (0, i, 0)),
        out_shape=jax.ShapeDtypeStruct((TOT_LEN, BATCH, _D), jnp.float32),
    )(prefix, gathered, suffix_t)


def kernel(label, prefix, suffix, ctx):
    label32 = label.astype(jnp.int32).reshape(BATCH)
    ctx2 = ctx.reshape(N_CLS, _CTX_W)
    suffix_t = suffix.transpose(1, 0, 2)
    gathered = _sc_gather_kernel()(label32, ctx2)
    out_t = _tc_assemble_call(prefix, gathered, suffix_t)
    return out_t.transpose(1, 0, 2)


# speedup vs baseline: 2.6425x; 1.0094x over previous
import functools

import jax
import jax.numpy as jnp
from jax import lax
from jax.experimental import pallas as pl
from jax.experimental.pallas import tpu as pltpu
from jax.experimental.pallas import tpu_sc as plsc

N_CLS = 1000
N_CTX = 16
CTX_DIM = 512
BATCH = 4096
SUF_LEN = 60
TOT_LEN = 1 + N_CTX + SUF_LEN

_D = CTX_DIM
_CTX_W = N_CTX * _D

_NC = 2
_NS = 16
_NW = _NC * _NS
_BPW = BATCH // _NW
_CH = 8
_NCHUNK = _BPW // _CH


def _sc_gather_kernel():
    mesh = plsc.VectorSubcoreMesh(core_axis_name="c", subcore_axis_name="s")

    @functools.partial(
        pl.kernel,
        mesh=mesh,
        out_type=jax.ShapeDtypeStruct((BATCH, _CTX_W), jnp.float32),
        scratch_types=[
            pltpu.VMEM((_BPW,), jnp.int32),
            pltpu.VMEM((_CH, _CTX_W), jnp.float32),
            pltpu.SemaphoreType.DMA,
        ],
    )
    def k(label_hbm, ctx_hbm, out_hbm, idx_v, buf, sem):
        wid = lax.axis_index("s") * _NC + lax.axis_index("c")
        base = wid * _BPW

        pltpu.sync_copy(label_hbm.at[pl.ds(base, _BPW)], idx_v.at[...])

        def body(c, carry):
            pltpu.async_copy(
                ctx_hbm.at[idx_v.at[pl.ds(c * _CH, _CH)]], buf, sem).wait()
            pltpu.sync_copy(buf.at[...], out_hbm.at[pl.ds(base + c * _CH, _CH)])
            return carry

        lax.fori_loop(0, _NCHUNK, body, 0)

    return k


_BB = 64


def _tc_assemble(pref_ref, gath_ref, suf_ref, out_ref):
    out_ref[0, :, :] = pref_ref[:, 0, :]
    for t in range(N_CTX):
        out_ref[1 + t, :, :] = gath_ref[:, t * _D:(t + 1) * _D]
    out_ref[1 + N_CTX:, :, :] = suf_ref[...]


def _tc_assemble_call(prefix, gathered, suffix_t):
    return pl.pallas_call(
        _tc_assemble,
        grid=(BATCH // _BB,),
        in_specs=[
            pl.BlockSpec((_BB, 1, _D), lambda i: (i, 0, 0)),
            pl.BlockSpec((_BB, _CTX_W), lambda i: (i, 0)),
            pl.BlockSpec((SUF_LEN, _BB, _D), lambda i: (0, i, 0)),
        ],
        out_specs=pl.BlockSpec((TOT_LEN, _BB, _D), lambda i: (0, i, 0)),
        out_shape=jax.ShapeDtypeStruct((TOT_LEN, BATCH, _D), jnp.float32),
    )(prefix, gathered, suffix_t)


def kernel(label, prefix, suffix, ctx):
    label32 = label.astype(jnp.int32).reshape(BATCH)
    ctx2 = ctx.reshape(N_CLS, _CTX_W)
    suffix_t = suffix.transpose(1, 0, 2)
    gathered = _sc_gather_kernel()(label32, ctx2)
    out_t = _tc_assemble_call(prefix, gathered, suffix_t)
    return out_t.transpose(1, 0, 2)
